# Initial kernel scaffold; baseline (speedup 1.0000x reference)
import jax
import jax.numpy as jnp
from jax.experimental import pallas as pl


def kernel(x, edge_index, W0, b0, W1, b1, W2, b2, W3, b3):
    raise NotImplementedError("write your pallas kernel here")


# speedup vs baseline: 14.6685x; 14.6685x over previous
import functools

import jax
import jax.numpy as jnp
from jax import lax
from jax.experimental import pallas as pl
from jax.experimental.pallas import tpu as pltpu
from jax.experimental.pallas import tpu_sc as plsc

N = 50000
E = 800000
D_IN = 128
H = 48
HH = 24
NC = 2
NS = 16
CHUNK = 128
NCH = 392
BCH = 56
E_PAD = NS * NCH * CHUNK
ACC_ROWS = 50048
DEG_ROWS = 51200
BN = 2000
GRID = N // BN

_mesh = plsc.VectorSubcoreMesh(
    core_axis_name="c", subcore_axis_name="s", num_cores=NC, num_subcores=NS)
_sc_params = pltpu.CompilerParams(use_tc_tiling_on_sc=False)


@functools.partial(
    pl.kernel,
    out_type=jax.ShapeDtypeStruct((NC, DEG_ROWS), jnp.float32),
    mesh=_mesh,
    scratch_types=[
        pltpu.VMEM((NCH // 2, CHUNK), jnp.int32),
        pltpu.VMEM((CHUNK,), jnp.float32),
        pltpu.VMEM_SHARED((DEG_ROWS,), jnp.float32),
    ],
    compiler_params=_sc_params,
)
def _deg_kernel(dstT_hbm, zeros_hbm, out_hbm, dst_v, ones_v, acc_sh):
    c = lax.axis_index("c")
    s = lax.axis_index("s")
    half = NCH // 2
    pltpu.sync_copy(dstT_hbm.at[c, s], dst_v)
    for k in range(CHUNK // 16):
        ones_v[pl.ds(16 * k, 16)] = jnp.ones((16,), jnp.float32)
    rz = DEG_ROWS // NS
    pltpu.sync_copy(zeros_hbm.at[pl.ds(s * rz, rz)], acc_sh.at[pl.ds(s * rz, rz)])
    plsc.subcore_barrier()

    def body(g, carry):
        pltpu.sync_copy(ones_v, acc_sh.at[dst_v.at[g]], add=True)
        return carry

    lax.fori_loop(0, half, body, 0)
    plsc.subcore_barrier()
    pltpu.sync_copy(acc_sh.at[pl.ds(s * rz, rz)], out_hbm.at[c, pl.ds(s * rz, rz)])


@functools.partial(
    pl.kernel,
    out_type=jax.ShapeDtypeStruct((NC, ACC_ROWS, HH), jnp.float32),
    mesh=_mesh,
    scratch_types=[
        pltpu.VMEM((BCH, CHUNK), jnp.int32),
        pltpu.VMEM((BCH, CHUNK), jnp.int32),
        pltpu.VMEM((CHUNK, HH), jnp.float32),
        pltpu.VMEM_SHARED((ACC_ROWS, HH), jnp.float32),
    ],
    compiler_params=_sc_params,
)
def _prop_kernel(u_hbm, srcT_hbm, dstT_hbm, zeros_hbm, out_hbm,
                 src_v, dst_v, rows_v, acc_sh):
    c = lax.axis_index("c")
    s = lax.axis_index("s")
    rz = ACC_ROWS // NS
    pltpu.sync_copy(zeros_hbm.at[pl.ds(s * rz, rz)], acc_sh.at[pl.ds(s * rz, rz)])
    plsc.subcore_barrier()

    def outer(b, carry):
        pltpu.sync_copy(srcT_hbm.at[c, s, pl.ds(b * BCH, BCH)], src_v)
        pltpu.sync_copy(dstT_hbm.at[s, pl.ds(b * BCH, BCH)], dst_v)

        def body(g, carry2):
            pltpu.sync_copy(u_hbm.at[src_v.at[g]], rows_v)
            pltpu.sync_copy(rows_v, acc_sh.at[dst_v.at[g]], add=True)
            return carry2

        return lax.fori_loop(0, BCH, body, carry)

    lax.fori_loop(0, NCH // BCH, outer, 0)
    plsc.subcore_barrier()
    pltpu.sync_copy(acc_sh.at[pl.ds(s * rz, rz)], out_hbm.at[c, pl.ds(s * rz, rz)])


def _tc1_body(x_r, deg_r, w0_r, b0_r, w1_r, h0_r, u1_r, dis_r):
    deg = deg_r[:, 0] + deg_r[:, 1] + 1.0
    dis = lax.rsqrt(deg)
    h0 = jnp.dot(x_r[...], w0_r[...], preferred_element_type=jnp.float32)
    h0 = h0 + b0_r[0, :]
    h0_r[...] = h0
    u = jnp.dot(h0, w1_r[...], preferred_element_type=jnp.float32)
    u = u * dis[:, None]
    u1_r[0] = u[:, :HH]
    u1_r[1] = u[:, HH:]
    dis_r[...] = dis[:, None]


def _tc_mid_body(acc_r, u_r, dis_r, b_r, w_r, h_r, un_r, *, relu):
    s0 = acc_r[0] + u_r[0]
    s1 = acc_r[1] + u_r[1]
    sfull = jnp.concatenate([s0, s1], axis=-1)
    dis = dis_r[...]
    h = sfull * dis + b_r[0, :]
    if relu:
        h = jnp.maximum(h, 0.0)
    h_r[...] = h
    un = jnp.dot(h, w_r[...], preferred_element_type=jnp.float32) * dis
    un_r[0] = un[:, :HH]
    un_r[1] = un[:, HH:]


def _tc_fin_body(acc_r, u_r, dis_r, b_r, h_r):
    s0 = acc_r[0] + u_r[0]
    s1 = acc_r[1] + u_r[1]
    sfull = jnp.concatenate([s0, s1], axis=-1)
    h_r[...] = sfull * dis_r[...] + b_r[0, :]


def _row_spec(w):
    return pl.BlockSpec((BN, w), lambda i: (i, 0))


def _half_spec():
    return pl.BlockSpec((2, BN, HH), lambda i: (0, i, 0))


def _full_spec(a, b):
    return pl.BlockSpec((a, b), lambda i: (0, 0))


_tc1 = pl.pallas_call(
    _tc1_body,
    grid=(GRID,),
    in_specs=[
        _row_spec(D_IN),
        pl.BlockSpec((BN, 2), lambda i: (i, 0)),
        _full_spec(D_IN, H),
        _full_spec(1, H),
        _full_spec(H, H),
    ],
    out_specs=[_row_spec(H), _half_spec(), _row_spec(1)],
    out_shape=[
        jax.ShapeDtypeStruct((N, H), jnp.float32),
        jax.ShapeDtypeStruct((2, N, HH), jnp.float32),
        jax.ShapeDtypeStruct((N, 1), jnp.float32),
    ],
)

def _make_mid(relu):
    return pl.pallas_call(
        functools.partial(_tc_mid_body, relu=relu),
        grid=(GRID,),
        in_specs=[
            _half_spec(),
            _half_spec(),
            _row_spec(1),
            _full_spec(1, H),
            _full_spec(H, H),
        ],
        out_specs=[_row_spec(H), _half_spec()],
        out_shape=[
            jax.ShapeDtypeStruct((N, H), jnp.float32),
            jax.ShapeDtypeStruct((2, N, HH), jnp.float32),
        ],
    )

_tc_mid = _make_mid(True)

_tc_fin = pl.pallas_call(
    _tc_fin_body,
    grid=(GRID,),
    in_specs=[_half_spec(), _half_spec(), _row_spec(1), _full_spec(1, H)],
    out_specs=_row_spec(H),
    out_shape=jax.ShapeDtypeStruct((N, H), jnp.float32),
)


def kernel(x, edge_index, W0, b0, W1, b1, W2, b2, W3, b3):
    src = edge_index[0]
    dst = edge_index[1]
    pad = E_PAD - E
    srcp = jnp.concatenate([src, jnp.zeros((pad,), jnp.int32)])
    dstp = jnp.concatenate([dst, jnp.full((pad,), N, jnp.int32)])
    srcT0 = srcp.reshape(NS, NCH, CHUNK)
    srcT = jnp.stack([srcT0, srcT0 + N])
    dstT = dstp.reshape(NS, NCH, CHUNK)
    dstT_deg = dstp.reshape(NC, NS, NCH // 2, CHUNK)

    zeros_deg = jnp.zeros((DEG_ROWS,), jnp.float32)
    zeros_acc = jnp.zeros((ACC_ROWS, HH), jnp.float32)

    deg_parts = _deg_kernel(dstT_deg, zeros_deg)
    deg2 = deg_parts[:, :N].T

    b0r = b0.reshape(1, H)
    b1r = b1.reshape(1, H)
    b2r = b2.reshape(1, H)
    b3r = jnp.concatenate([b3, jnp.zeros((H - 40,), jnp.float32)]).reshape(1, H)
    W3p = jnp.concatenate([W3, jnp.zeros((H, H - 40), jnp.float32)], axis=1)

    h0, u1, dis = _tc1(x, deg2, W0, b0r, W1)
    acc1 = _prop_kernel(u1.reshape(NC * N, HH), srcT, dstT, zeros_acc)
    h1, u2 = _tc_mid(acc1, u1, dis, b1r, W2)
    acc2 = _prop_kernel(u2.reshape(NC * N, HH), srcT, dstT, zeros_acc)
    h2, u3 = _tc_mid(acc2, u2, dis, b2r, W3p)
    acc3 = _prop_kernel(u3.reshape(NC * N, HH), srcT, dstT, zeros_acc)
    h3 = _tc_fin(acc3, u3, dis, b3r)

    return jnp.concatenate([h0, h1, h2, h3[:, :40]], axis=-1)


# speedup vs baseline: 22.6733x; 1.5457x over previous
import functools

import jax
import jax.numpy as jnp
from jax import lax
from jax.experimental import pallas as pl
from jax.experimental.pallas import tpu as pltpu
from jax.experimental.pallas import tpu_sc as plsc

N = 50000
E = 800000
D_IN = 128
H = 48
HH = 24
NC = 2
NS = 16
CHUNK = 128
NCH = 392
BCH = 28
NBUF = 4
E_PAD = NS * NCH * CHUNK
ACC_ROWS = 50048
DEG_ROWS = 51200
BN = 2000
GRID = N // BN

_mesh = plsc.VectorSubcoreMesh(
    core_axis_name="c", subcore_axis_name="s", num_cores=NC, num_subcores=NS)
_sc_params = pltpu.CompilerParams(use_tc_tiling_on_sc=False)


@functools.partial(
    pl.kernel,
    out_type=jax.ShapeDtypeStruct((NC, DEG_ROWS), jnp.float32),
    mesh=_mesh,
    scratch_types=[
        pltpu.VMEM((NCH // 2, CHUNK), jnp.int32),
        pltpu.VMEM((CHUNK,), jnp.float32),
        pltpu.VMEM_SHARED((DEG_ROWS,), jnp.float32),
    ],
    compiler_params=_sc_params,
)
def _deg_kernel(dstT_hbm, zeros_hbm, out_hbm, dst_v, ones_v, acc_sh):
    c = lax.axis_index("c")
    s = lax.axis_index("s")
    half = NCH // 2
    pltpu.sync_copy(dstT_hbm.at[c, s], dst_v)
    for k in range(CHUNK // 16):
        ones_v[pl.ds(16 * k, 16)] = jnp.ones((16,), jnp.float32)
    rz = DEG_ROWS // NS
    pltpu.sync_copy(zeros_hbm.at[pl.ds(s * rz, rz)], acc_sh.at[pl.ds(s * rz, rz)])
    plsc.subcore_barrier()

    def body(g, carry):
        pltpu.sync_copy(ones_v, acc_sh.at[dst_v.at[g]], add=True)
        return carry

    lax.fori_loop(0, half, body, 0)
    plsc.subcore_barrier()
    pltpu.sync_copy(acc_sh.at[pl.ds(s * rz, rz)], out_hbm.at[c, pl.ds(s * rz, rz)])


@functools.partial(
    pl.kernel,
    out_type=jax.ShapeDtypeStruct((NC, ACC_ROWS, HH), jnp.float32),
    mesh=_mesh,
    scratch_types=[
        pltpu.VMEM((BCH, CHUNK), jnp.int32),
        pltpu.VMEM((BCH, CHUNK), jnp.int32),
        pltpu.VMEM((NBUF, CHUNK, HH), jnp.float32),
        pltpu.SemaphoreType.DMA,
        pltpu.SemaphoreType.DMA,
        pltpu.SemaphoreType.DMA,
        pltpu.SemaphoreType.DMA,
        pltpu.VMEM_SHARED((ACC_ROWS, HH), jnp.float32),
    ],
    compiler_params=_sc_params,
)
def _prop_kernel(u_hbm, srcT_hbm, dstT_hbm, zeros_hbm, out_hbm,
                 src_v, dst_v, rows_v, sem0, sem1, sem2, sem3, acc_sh):
    sems = (sem0, sem1, sem2, sem3)
    c = lax.axis_index("c")
    s = lax.axis_index("s")
    rz = ACC_ROWS // NS
    pltpu.sync_copy(zeros_hbm.at[pl.ds(s * rz, rz)], acc_sh.at[pl.ds(s * rz, rz)])
    plsc.subcore_barrier()

    for k in range(NCH // BCH):
        pltpu.sync_copy(srcT_hbm.at[c, s, pl.ds(k * BCH, BCH)], src_v)
        pltpu.sync_copy(dstT_hbm.at[s, pl.ds(k * BCH, BCH)], dst_v)
        for b in range(NBUF - 1):
            pltpu.async_copy(u_hbm.at[src_v.at[b]], rows_v.at[b], sems[b])

        def body(i, carry):
            jj = i * NBUF
            for b in range(NBUF):
                j = jj + b
                pf = j + NBUF - 1
                bpf = (b + NBUF - 1) % NBUF

                @pl.when(pf < BCH)
                def _():
                    pltpu.async_copy(u_hbm.at[src_v.at[pf]], rows_v.at[bpf],
                                     sems[bpf])

                pltpu.make_async_copy(u_hbm.at[src_v.at[j]], rows_v.at[b],
                                      sems[b]).wait()
                pltpu.sync_copy(rows_v.at[b], acc_sh.at[dst_v.at[j]], add=True)
            return carry

        lax.fori_loop(0, BCH // NBUF, body, 0)
    plsc.subcore_barrier()
    pltpu.sync_copy(acc_sh.at[pl.ds(s * rz, rz)], out_hbm.at[c, pl.ds(s * rz, rz)])


def _tc1_body(x_r, deg_r, w0_r, b0_r, w1_r, h0_r, u1_r, dis_r):
    deg = deg_r[:, 0] + deg_r[:, 1] + 1.0
    dis = lax.rsqrt(deg)
    h0 = jnp.dot(x_r[...], w0_r[...], preferred_element_type=jnp.float32)
    h0 = h0 + b0_r[0, :]
    h0_r[...] = h0
    u = jnp.dot(h0, w1_r[...], preferred_element_type=jnp.float32)
    u = u * dis[:, None]
    u1_r[0] = u[:, :HH]
    u1_r[1] = u[:, HH:]
    dis_r[...] = dis[:, None]


def _tc_mid_body(acc_r, u_r, dis_r, b_r, w_r, h_r, un_r, *, relu):
    s0 = acc_r[0] + u_r[0]
    s1 = acc_r[1] + u_r[1]
    sfull = jnp.concatenate([s0, s1], axis=-1)
    dis = dis_r[...]
    h = sfull * dis + b_r[0, :]
    if relu:
        h = jnp.maximum(h, 0.0)
    h_r[...] = h
    un = jnp.dot(h, w_r[...], preferred_element_type=jnp.float32) * dis
    un_r[0] = un[:, :HH]
    un_r[1] = un[:, HH:]


def _tc_fin_body(acc_r, u_r, dis_r, b_r, h_r):
    s0 = acc_r[0] + u_r[0]
    s1 = acc_r[1] + u_r[1]
    sfull = jnp.concatenate([s0, s1], axis=-1)
    h_r[...] = sfull * dis_r[...] + b_r[0, :]


def _row_spec(w):
    return pl.BlockSpec((BN, w), lambda i: (i, 0))


def _half_spec():
    return pl.BlockSpec((2, BN, HH), lambda i: (0, i, 0))


def _full_spec(a, b):
    return pl.BlockSpec((a, b), lambda i: (0, 0))


_tc1 = pl.pallas_call(
    _tc1_body,
    grid=(GRID,),
    in_specs=[
        _row_spec(D_IN),
        pl.BlockSpec((BN, 2), lambda i: (i, 0)),
        _full_spec(D_IN, H),
        _full_spec(1, H),
        _full_spec(H, H),
    ],
    out_specs=[_row_spec(H), _half_spec(), _row_spec(1)],
    out_shape=[
        jax.ShapeDtypeStruct((N, H), jnp.float32),
        jax.ShapeDtypeStruct((2, N, HH), jnp.float32),
        jax.ShapeDtypeStruct((N, 1), jnp.float32),
    ],
)

def _make_mid(relu):
    return pl.pallas_call(
        functools.partial(_tc_mid_body, relu=relu),
        grid=(GRID,),
        in_specs=[
            _half_spec(),
            _half_spec(),
            _row_spec(1),
            _full_spec(1, H),
            _full_spec(H, H),
        ],
        out_specs=[_row_spec(H), _half_spec()],
        out_shape=[
            jax.ShapeDtypeStruct((N, H), jnp.float32),
            jax.ShapeDtypeStruct((2, N, HH), jnp.float32),
        ],
    )

_tc_mid = _make_mid(True)

_tc_fin = pl.pallas_call(
    _tc_fin_body,
    grid=(GRID,),
    in_specs=[_half_spec(), _half_spec(), _row_spec(1), _full_spec(1, H)],
    out_specs=_row_spec(H),
    out_shape=jax.ShapeDtypeStruct((N, H), jnp.float32),
)


def kernel(x, edge_index, W0, b0, W1, b1, W2, b2, W3, b3):
    src = edge_index[0]
    dst = edge_index[1]
    pad = E_PAD - E
    srcp = jnp.concatenate([src, jnp.zeros((pad,), jnp.int32)])
    dstp = jnp.concatenate([dst, jnp.full((pad,), N, jnp.int32)])
    srcT0 = srcp.reshape(NS, NCH, CHUNK)
    srcT = jnp.stack([srcT0, srcT0 + N])
    dstT = dstp.reshape(NS, NCH, CHUNK)
    dstT_deg = dstp.reshape(NC, NS, NCH // 2, CHUNK)

    zeros_deg = jnp.zeros((DEG_ROWS,), jnp.float32)
    zeros_acc = jnp.zeros((ACC_ROWS, HH), jnp.float32)

    deg_parts = _deg_kernel(dstT_deg, zeros_deg)
    deg2 = deg_parts[:, :N].T

    b0r = b0.reshape(1, H)
    b1r = b1.reshape(1, H)
    b2r = b2.reshape(1, H)
    b3r = jnp.concatenate([b3, jnp.zeros((H - 40,), jnp.float32)]).reshape(1, H)
    W3p = jnp.concatenate([W3, jnp.zeros((H, H - 40), jnp.float32)], axis=1)

    h0, u1, dis = _tc1(x, deg2, W0, b0r, W1)
    acc1 = _prop_kernel(u1.reshape(NC * N, HH), srcT, dstT, zeros_acc)
    h1, u2 = _tc_mid(acc1, u1, dis, b1r, W2)
    acc2 = _prop_kernel(u2.reshape(NC * N, HH), srcT, dstT, zeros_acc)
    h2, u3 = _tc_mid(acc2, u2, dis, b2r, W3p)
    acc3 = _prop_kernel(u3.reshape(NC * N, HH), srcT, dstT, zeros_acc)
    h3 = _tc_fin(acc3, u3, dis, b3r)

    return jnp.concatenate([h0, h1, h2, h3[:, :40]], axis=-1)


# speedup vs baseline: 25.7373x; 1.1351x over previous
import functools

import jax
import jax.numpy as jnp
from jax import lax
from jax.experimental import pallas as pl
from jax.experimental.pallas import tpu as pltpu
from jax.experimental.pallas import tpu_sc as plsc

N = 50000
E = 800000
D_IN = 128
H = 48
C = 40
OUT = 3 * H + C
HH = 24
NC = 2
NS = 16
CHUNK = 128
NCH = 392
BCH = 28
NBUF = 4
E_PAD = NS * NCH * CHUNK
ACC_ROWS = 50048
DEG_ROWS = 51200
BN = 2000
GRID = N // BN

_mesh = plsc.VectorSubcoreMesh(
    core_axis_name="c", subcore_axis_name="s", num_cores=NC, num_subcores=NS)
_sc_params = pltpu.CompilerParams(use_tc_tiling_on_sc=False)


@functools.partial(
    pl.kernel,
    out_type=jax.ShapeDtypeStruct((NC, DEG_ROWS), jnp.float32),
    mesh=_mesh,
    scratch_types=[
        pltpu.VMEM((NCH // 2, CHUNK), jnp.int32),
        pltpu.VMEM((CHUNK,), jnp.float32),
        pltpu.VMEM_SHARED((DEG_ROWS,), jnp.float32),
    ],
    compiler_params=_sc_params,
)
def _deg_kernel(dstT_hbm, zeros_hbm, out_hbm, dst_v, ones_v, acc_sh):
    c = lax.axis_index("c")
    s = lax.axis_index("s")
    half = NCH // 2
    pltpu.sync_copy(dstT_hbm.at[c, s], dst_v)
    for k in range(CHUNK // 16):
        ones_v[pl.ds(16 * k, 16)] = jnp.ones((16,), jnp.float32)
    rz = DEG_ROWS // NS
    pltpu.sync_copy(zeros_hbm.at[pl.ds(s * rz, rz)], acc_sh.at[pl.ds(s * rz, rz)])
    plsc.subcore_barrier()

    def body(g, carry):
        pltpu.sync_copy(ones_v, acc_sh.at[dst_v.at[g]], add=True)
        return carry

    lax.fori_loop(0, half, body, 0)
    plsc.subcore_barrier()
    pltpu.sync_copy(acc_sh.at[pl.ds(s * rz, rz)], out_hbm.at[c, pl.ds(s * rz, rz)])


@functools.partial(
    pl.kernel,
    out_type=jax.ShapeDtypeStruct((NC, ACC_ROWS, HH), jnp.float32),
    mesh=_mesh,
    scratch_types=[
        pltpu.VMEM((BCH, CHUNK), jnp.int32),
        pltpu.VMEM((BCH, CHUNK), jnp.int32),
        pltpu.VMEM((NBUF, CHUNK, HH), jnp.float32),
        pltpu.SemaphoreType.DMA,
        pltpu.SemaphoreType.DMA,
        pltpu.SemaphoreType.DMA,
        pltpu.SemaphoreType.DMA,
        pltpu.VMEM_SHARED((ACC_ROWS, HH), jnp.float32),
    ],
    compiler_params=_sc_params,
)
def _prop_kernel(u_hbm, srcT_hbm, dstT_hbm, zeros_hbm, out_hbm,
                 src_v, dst_v, rows_v, sem0, sem1, sem2, sem3, acc_sh):
    sems = (sem0, sem1, sem2, sem3)
    c = lax.axis_index("c")
    s = lax.axis_index("s")
    rz = ACC_ROWS // NS
    pltpu.sync_copy(zeros_hbm.at[pl.ds(s * rz, rz)], acc_sh.at[pl.ds(s * rz, rz)])
    plsc.subcore_barrier()

    for k in range(NCH // BCH):
        pltpu.sync_copy(srcT_hbm.at[s, pl.ds(k * BCH, BCH)], src_v)
        pltpu.sync_copy(dstT_hbm.at[s, pl.ds(k * BCH, BCH)], dst_v)
        for b in range(NBUF - 1):
            pltpu.async_copy(u_hbm.at[pl.ds(c * N, N)].at[src_v.at[b]], rows_v.at[b], sems[b])

        def body(i, carry):
            jj = i * NBUF
            for b in range(NBUF):
                j = jj + b
                pf = j + NBUF - 1
                bpf = (b + NBUF - 1) % NBUF

                @pl.when(pf < BCH)
                def _():
                    pltpu.async_copy(u_hbm.at[pl.ds(c * N, N)].at[src_v.at[pf]], rows_v.at[bpf],
                                     sems[bpf])

                pltpu.make_async_copy(u_hbm.at[pl.ds(c * N, N)].at[src_v.at[j]], rows_v.at[b],
                                      sems[b]).wait()
                pltpu.sync_copy(rows_v.at[b], acc_sh.at[dst_v.at[j]], add=True)
            return carry

        lax.fori_loop(0, BCH // NBUF, body, 0)
    plsc.subcore_barrier()
    pltpu.sync_copy(acc_sh.at[pl.ds(s * rz, rz)], out_hbm.at[c, pl.ds(s * rz, rz)])


def _tc1_body(x_r, deg_r, w0_r, b0_r, w1_r, h0_r, u1_r, dis_r):
    dis = lax.rsqrt(deg_r[...] + 1.0)
    h0 = jnp.dot(x_r[...], w0_r[...], preferred_element_type=jnp.float32)
    h0 = h0 + b0_r[0, :]
    h0_r[...] = h0
    u = jnp.dot(h0, w1_r[...], preferred_element_type=jnp.float32)
    u = u * dis
    u1_r[0] = u[:, :HH]
    u1_r[1] = u[:, HH:]
    dis_r[...] = dis


def _tc_mid_body(acc_r, u_r, dis_r, b_r, w_r, h_r, un_r, *, relu):
    s0 = acc_r[0] + u_r[0]
    s1 = acc_r[1] + u_r[1]
    sfull = jnp.concatenate([s0, s1], axis=-1)
    dis = dis_r[...]
    h = sfull * dis + b_r[0, :]
    if relu:
        h = jnp.maximum(h, 0.0)
    h_r[...] = h
    un = jnp.dot(h, w_r[...], preferred_element_type=jnp.float32) * dis
    un_r[0] = un[:, :HH]
    un_r[1] = un[:, HH:]


def _tc_fin_body(h0_r, h1_r, h2_r, acc_r, u_r, dis_r, b_r, out_r):
    s0 = acc_r[0] + u_r[0]
    s1 = acc_r[1] + u_r[1]
    sfull = jnp.concatenate([s0, s1], axis=-1)
    h3 = sfull * dis_r[...] + b_r[0, :]
    out_r[...] = jnp.concatenate(
        [h0_r[...], h1_r[...], h2_r[...], h3[:, :C]], axis=-1)


def _row_spec(w):
    return pl.BlockSpec((BN, w), lambda i: (i, 0))


def _half_spec():
    return pl.BlockSpec((2, BN, HH), lambda i: (0, i, 0))


def _full_spec(a, b):
    return pl.BlockSpec((a, b), lambda i: (0, 0))


_tc1 = pl.pallas_call(
    _tc1_body,
    grid=(GRID,),
    in_specs=[
        _row_spec(D_IN),
        _row_spec(1),
        _full_spec(D_IN, H),
        _full_spec(1, H),
        _full_spec(H, H),
    ],
    out_specs=[_row_spec(H), _half_spec(), _row_spec(1)],
    out_shape=[
        jax.ShapeDtypeStruct((N, H), jnp.float32),
        jax.ShapeDtypeStruct((2, N, HH), jnp.float32),
        jax.ShapeDtypeStruct((N, 1), jnp.float32),
    ],
)

def _make_mid(relu):
    return pl.pallas_call(
        functools.partial(_tc_mid_body, relu=relu),
        grid=(GRID,),
        in_specs=[
            _half_spec(),
            _half_spec(),
            _row_spec(1),
            _full_spec(1, H),
            _full_spec(H, H),
        ],
        out_specs=[_row_spec(H), _half_spec()],
        out_shape=[
            jax.ShapeDtypeStruct((N, H), jnp.float32),
            jax.ShapeDtypeStruct((2, N, HH), jnp.float32),
        ],
    )

_tc_mid = _make_mid(True)

_tc_fin = pl.pallas_call(
    _tc_fin_body,
    grid=(GRID,),
    in_specs=[_row_spec(H), _row_spec(H), _row_spec(H),
              _half_spec(), _half_spec(), _row_spec(1), _full_spec(1, H)],
    out_specs=_row_spec(OUT),
    out_shape=jax.ShapeDtypeStruct((N, OUT), jnp.float32),
)


def kernel(x, edge_index, W0, b0, W1, b1, W2, b2, W3, b3):
    src = edge_index[0]
    dst = edge_index[1]
    pad = E_PAD - E
    srcp = jnp.concatenate([src, jnp.zeros((pad,), jnp.int32)])
    dstp = jnp.concatenate([dst, jnp.full((pad,), N, jnp.int32)])
    srcT = srcp.reshape(NS, NCH, CHUNK)
    dstT = dstp.reshape(NS, NCH, CHUNK)
    dstT_deg = dstp.reshape(NC, NS, NCH // 2, CHUNK)

    zeros_deg = jnp.zeros((DEG_ROWS,), jnp.float32)
    zeros_acc = jnp.zeros((ACC_ROWS, HH), jnp.float32)

    deg_parts = _deg_kernel(dstT_deg, zeros_deg)
    deg2 = (deg_parts[0, :N] + deg_parts[1, :N]).reshape(N, 1)

    b0r = b0.reshape(1, H)
    b1r = b1.reshape(1, H)
    b2r = b2.reshape(1, H)
    b3r = jnp.concatenate([b3, jnp.zeros((H - C,), jnp.float32)]).reshape(1, H)
    W3p = jnp.concatenate([W3, jnp.zeros((H, H - C), jnp.float32)], axis=1)

    h0, u1, dis = _tc1(x, deg2, W0, b0r, W1)
    acc1 = _prop_kernel(u1.reshape(NC * N, HH), srcT, dstT, zeros_acc)
    h1, u2 = _tc_mid(acc1, u1, dis, b1r, W2)
    acc2 = _prop_kernel(u2.reshape(NC * N, HH), srcT, dstT, zeros_acc)
    h2, u3 = _tc_mid(acc2, u2, dis, b2r, W3p)
    acc3 = _prop_kernel(u3.reshape(NC * N, HH), srcT, dstT, zeros_acc)
    return _tc_fin(h0, h1, h2, acc3, u3, dis, b3r)
